# baseline (speedup 1.0000x reference)
import jax
import jax.numpy as jnp
from jax.experimental import pallas as pl
from jax.experimental.pallas import tpu as pltpu

_N = 4096
_R = 512
_G = _N // _R
_RM = 128
_GM = _N // _RM


def _leaky(z):
    return jnp.where(z >= 0, z, 0.2 * z)


def _dot(a, b):
    return jnp.dot(a, b, preferred_element_type=jnp.float32)


def _soft_assign(z, cluster):
    zn = jnp.sum(z * z, axis=1, keepdims=True)
    cn = jnp.sum(cluster * cluster, axis=1)[None, :]
    d2 = zn + cn - 2.0 * _dot(z, cluster.T)
    q = 1.0 / (1.0 + d2)
    return q / jnp.sum(q, axis=1, keepdims=True)


def _pre_kernel(x_ref, w0, b0, w1, b1, w2, b2, w3, b3,
                gw0, gw1, gw2, cl, zae_out, q1_out, v0_out):
    x = x_ref[...]
    z = _leaky(_dot(x, w0[...]) + b0[...])
    z = _leaky(_dot(z, w1[...]) + b1[...])
    z = _leaky(_dot(z, w2[...]) + b2[...])
    zae = _dot(z, w3[...]) + b3[...]
    zae_out[...] = zae
    q1_out[...] = _soft_assign(zae, cl[...])
    wg = _dot(_dot(gw0[...], gw1[...]), gw2[...])
    v0_out[...] = _dot(x, wg)


def _mega_kernel(adj_ref, v0_ref, zae_ref, a_ref, gamma_ref,
                 zs_out, zt_out, t3_out,
                 adjv, va, vb):
    s = pl.program_id(0)
    r = pl.program_id(1)
    rs = pl.ds(r * _RM, _RM)

    @pl.when(s == 0)
    def _():
        adjv[rs, :] = adj_ref[...].astype(jnp.bfloat16)
        va[rs, :] = v0_ref[rs, :]

    adjr = adjv[rs, :]

    @pl.when(s == 1)
    def _():
        vb[rs, :] = _dot(adjr, va[...].astype(jnp.bfloat16))

    @pl.when(s == 2)
    def _():
        va[rs, :] = _dot(adjr, vb[...].astype(jnp.bfloat16))

    @pl.when(s == 3)
    def _():
        zs_r = _dot(adjr, va[...].astype(jnp.bfloat16))
        zs_out[rs, :] = zs_r
        a_r = a_ref[rs, :]
        vb[rs, :] = a_r * zae_ref[rs, :] + (1.0 - a_r) * zs_r

    @pl.when(s == 4)
    def _():
        va[rs, :] = _dot(adjr, vb[...].astype(jnp.bfloat16))

    @pl.when(s == 5)
    def _():
        zl_r = va[rs, :]
        m = jnp.full((_RM, 1), -jnp.inf, dtype=jnp.float32)
        den = jnp.zeros((_RM, 1), dtype=jnp.float32)
        acc = jnp.zeros((_RM, 32), dtype=jnp.float32)
        for c in range(4):
            zl_c = va[pl.ds(c * (_N // 4), _N // 4), :]
            sc = _dot(zl_r, zl_c.T)
            m_new = jnp.maximum(m, jnp.max(sc, axis=1, keepdims=True))
            alpha = jnp.exp(m - m_new)
            pch = jnp.exp(sc - m_new)
            den = den * alpha + jnp.sum(pch, axis=1, keepdims=True)
            acc = acc * alpha + _dot(pch, zl_c)
            m = m_new
        zt_r = gamma_ref[0, 0] * (acc / den) + zl_r
        zt_out[rs, :] = zt_r
        vb[rs, :] = zt_r

    @pl.when(s == 6)
    def _():
        va[rs, :] = _dot(adjr, vb[...].astype(jnp.bfloat16))

    @pl.when(s == 7)
    def _():
        vb[rs, :] = _dot(adjr, va[...].astype(jnp.bfloat16))

    @pl.when(s == 8)
    def _():
        t3_out[rs, :] = _dot(adjr, vb[...].astype(jnp.bfloat16))


def _tail_kernel(zt_ref, t3_ref, zs_ref,
                 dw0, db0, dw1, db1, dw2, db2, dw3, db3,
                 gw0, gw1, gw2, cl,
                 xhat_out, zhat_out, q_out, q2_out, tp_out):
    zt = zt_ref[...]
    d = _leaky(_dot(zt, dw0[...]) + db0[...])
    d = _leaky(_dot(d, dw1[...]) + db1[...])
    d = _leaky(_dot(d, dw2[...]) + db2[...])
    xhat_out[...] = _dot(d, dw3[...]) + db3[...]
    ug = _dot(_dot(gw0[...], gw1[...]), gw2[...])
    t3 = t3_ref[...]
    zhat_out[...] = _dot(t3, ug)
    tp_out[...] = _dot(t3, _dot(ug, ug.T))
    q_out[...] = _soft_assign(zt, cl[...])
    q2_out[...] = _soft_assign(zs_ref[...], cl[...])


def _adjhat_kernel(zs_r_ref, zs_ref, tp_ref, t3_ref, o_ref):
    a1 = _dot(zs_r_ref[...], zs_ref[...].T)
    a2 = _dot(tp_ref[...], t3_ref[...].T)
    o_ref[...] = jax.nn.sigmoid(a1) + jax.nn.sigmoid(a2)


def _full(arr):
    nd = arr.ndim
    return pl.BlockSpec(arr.shape, lambda i, _n=nd: (0,) * _n)


def _row(last):
    return pl.BlockSpec((_R, last), lambda i: (i, 0))


def _sds(shape):
    return jax.ShapeDtypeStruct(shape, jnp.float32)


def kernel(x, adj, params):
    p = params
    b = {k: p[k].reshape(1, -1) for k in p if k.startswith('ae_') and '_b' in k}
    gamma = p['gamma'].reshape(1, 1)
    cl = p['cluster']

    zae, q1, v0 = pl.pallas_call(
        _pre_kernel,
        grid=(_G,),
        in_specs=[_row(512),
                  _full(p['ae_enc_w0']), _full(b['ae_enc_b0']),
                  _full(p['ae_enc_w1']), _full(b['ae_enc_b1']),
                  _full(p['ae_enc_w2']), _full(b['ae_enc_b2']),
                  _full(p['ae_enc_w3']), _full(b['ae_enc_b3']),
                  _full(p['gae_enc_w0']), _full(p['gae_enc_w1']),
                  _full(p['gae_enc_w2']), _full(cl)],
        out_specs=[_row(32), _row(10), _row(32)],
        out_shape=[_sds((_N, 32)), _sds((_N, 10)), _sds((_N, 32))],
    )(x, p['ae_enc_w0'], b['ae_enc_b0'], p['ae_enc_w1'], b['ae_enc_b1'],
      p['ae_enc_w2'], b['ae_enc_b2'], p['ae_enc_w3'], b['ae_enc_b3'],
      p['gae_enc_w0'], p['gae_enc_w1'], p['gae_enc_w2'], cl)

    def cfull(shape):
        return pl.BlockSpec(shape, lambda s_, r_: (0,) * len(shape))

    adj_spec = pl.BlockSpec(
        (_RM, _N), lambda s_, r_: (jnp.where(s_ == 0, r_, _GM - 1), 0))

    mega_ins = [v0, zae, p['a'], gamma]
    zs, zt, t3 = pl.pallas_call(
        _mega_kernel,
        grid=(9, _GM),
        in_specs=[adj_spec] + [cfull(t.shape) for t in mega_ins],
        out_specs=[cfull((_N, 32)), cfull((_N, 32)), cfull((_N, 32))],
        out_shape=[_sds((_N, 32)), _sds((_N, 32)), _sds((_N, 32))],
        scratch_shapes=[pltpu.VMEM((_N, _N), jnp.bfloat16),
                        pltpu.VMEM((_N, 32), jnp.float32),
                        pltpu.VMEM((_N, 32), jnp.float32)],
    )(adj, *mega_ins)

    xhat, zhat, q, q2, tp = pl.pallas_call(
        _tail_kernel,
        grid=(_G,),
        in_specs=[_row(32), _row(32), _row(32),
                  _full(p['ae_dec_w0']), _full(b['ae_dec_b0']),
                  _full(p['ae_dec_w1']), _full(b['ae_dec_b1']),
                  _full(p['ae_dec_w2']), _full(b['ae_dec_b2']),
                  _full(p['ae_dec_w3']), _full(b['ae_dec_b3']),
                  _full(p['gae_dec_w0']), _full(p['gae_dec_w1']),
                  _full(p['gae_dec_w2']), _full(cl)],
        out_specs=[_row(512), _row(512), _row(10), _row(10), _row(32)],
        out_shape=[_sds((_N, 512)), _sds((_N, 512)), _sds((_N, 10)),
                   _sds((_N, 10)), _sds((_N, 32))],
    )(zt, t3, zs,
      p['ae_dec_w0'], b['ae_dec_b0'], p['ae_dec_w1'], b['ae_dec_b1'],
      p['ae_dec_w2'], b['ae_dec_b2'], p['ae_dec_w3'], b['ae_dec_b3'],
      p['gae_dec_w0'], p['gae_dec_w1'], p['gae_dec_w2'], cl)

    adj_hat = pl.pallas_call(
        _adjhat_kernel,
        grid=(_G,),
        in_specs=[_row(32), _full(zs), _row(32), _full(t3)],
        out_specs=_row(_N),
        out_shape=_sds((_N, _N)),
    )(zs, zs, tp, t3)

    return (xhat, zhat, adj_hat, zae, zs, q, q1, q2, zt)


# speedup vs baseline: 1.4046x; 1.4046x over previous
import jax
import jax.numpy as jnp
from jax.experimental import pallas as pl
from jax.experimental.pallas import tpu as pltpu

_N = 4096
_R = 512
_G = _N // _R
_RC = 128
_TC = _N // _RC
_RS = 512
_GS = _N // _RS
_T = _TC + 8 * _GS


def _leaky(z):
    return jnp.where(z >= 0, z, 0.2 * z)


def _dot(a, b):
    return jnp.dot(a, b, preferred_element_type=jnp.float32)


def _soft_assign(z, cluster):
    zn = jnp.sum(z * z, axis=1, keepdims=True)
    cn = jnp.sum(cluster * cluster, axis=1)[None, :]
    d2 = zn + cn - 2.0 * _dot(z, cluster.T)
    q = 1.0 / (1.0 + d2)
    return q / jnp.sum(q, axis=1, keepdims=True)


def _pre_kernel(x_ref, w0, b0, w1, b1, w2, b2, w3, b3,
                gw0, gw1, gw2, cl, zae_out, q1_out, v0_out):
    x = x_ref[...]
    z = _leaky(_dot(x, w0[...]) + b0[...])
    z = _leaky(_dot(z, w1[...]) + b1[...])
    z = _leaky(_dot(z, w2[...]) + b2[...])
    zae = _dot(z, w3[...]) + b3[...]
    zae_out[...] = zae
    q1_out[...] = _soft_assign(zae, cl[...])
    wg = _dot(_dot(gw0[...], gw1[...]), gw2[...])
    v0_out[...] = _dot(x, wg)


def _mega_kernel(adj_ref, v0_ref, zae_ref, a_ref, gamma_ref,
                 zs_out, zt_out, t3_out,
                 adjv, va, vb):
    t = pl.program_id(0)
    s = (t - _TC) // _GS
    r = (t - _TC) % _GS
    rs = pl.ds(r * _RS, _RS)

    @pl.when(t < _TC)
    def _():
        cs = pl.ds(t * _RC, _RC)
        adjv[cs, :] = adj_ref[...].astype(jnp.bfloat16)
        va[cs, :] = v0_ref[cs, :]

    @pl.when(s == 0)
    def _():
        vb[rs, :] = _dot(adjv[rs, :], va[...].astype(jnp.bfloat16))

    @pl.when(s == 1)
    def _():
        va[rs, :] = _dot(adjv[rs, :], vb[...].astype(jnp.bfloat16))

    @pl.when(s == 2)
    def _():
        zs_r = _dot(adjv[rs, :], va[...].astype(jnp.bfloat16))
        zs_out[rs, :] = zs_r
        a_r = a_ref[rs, :]
        vb[rs, :] = a_r * zae_ref[rs, :] + (1.0 - a_r) * zs_r

    @pl.when(s == 3)
    def _():
        va[rs, :] = _dot(adjv[rs, :], vb[...].astype(jnp.bfloat16))

    @pl.when(s == 4)
    def _():
        zl_r = va[rs, :]
        m = jnp.full((_RS, 1), -jnp.inf, dtype=jnp.float32)
        den = jnp.zeros((_RS, 1), dtype=jnp.float32)
        acc = jnp.zeros((_RS, 32), dtype=jnp.float32)
        for c in range(16):
            zl_c = va[pl.ds(c * (_N // 16), _N // 16), :]
            sc = _dot(zl_r, zl_c.T)
            m_new = jnp.maximum(m, jnp.max(sc, axis=1, keepdims=True))
            alpha = jnp.exp(m - m_new)
            pch = jnp.exp(sc - m_new)
            den = den * alpha + jnp.sum(pch, axis=1, keepdims=True)
            acc = acc * alpha + _dot(pch, zl_c)
            m = m_new
        zt_r = gamma_ref[0, 0] * (acc / den) + zl_r
        zt_out[rs, :] = zt_r
        vb[rs, :] = zt_r

    @pl.when(s == 5)
    def _():
        va[rs, :] = _dot(adjv[rs, :], vb[...].astype(jnp.bfloat16))

    @pl.when(s == 6)
    def _():
        vb[rs, :] = _dot(adjv[rs, :], va[...].astype(jnp.bfloat16))

    @pl.when(s == 7)
    def _():
        t3_out[rs, :] = _dot(adjv[rs, :], vb[...].astype(jnp.bfloat16))


def _tail_kernel(zt_ref, t3_ref, zs_ref,
                 dw0, db0, dw1, db1, dw2, db2, dw3, db3,
                 gw0, gw1, gw2, cl,
                 xhat_out, zhat_out, q_out, q2_out, tp_out):
    zt = zt_ref[...]
    d = _leaky(_dot(zt, dw0[...]) + db0[...])
    d = _leaky(_dot(d, dw1[...]) + db1[...])
    d = _leaky(_dot(d, dw2[...]) + db2[...])
    xhat_out[...] = _dot(d, dw3[...]) + db3[...]
    ug = _dot(_dot(gw0[...], gw1[...]), gw2[...])
    t3 = t3_ref[...]
    zhat_out[...] = _dot(t3, ug)
    tp_out[...] = _dot(t3, _dot(ug, ug.T))
    q_out[...] = _soft_assign(zt, cl[...])
    q2_out[...] = _soft_assign(zs_ref[...], cl[...])


def _adjhat_kernel(zs_r_ref, zs_ref, tp_ref, t3_ref, o_ref):
    a1 = _dot(zs_r_ref[...], zs_ref[...].T)
    a2 = _dot(tp_ref[...], t3_ref[...].T)
    o_ref[...] = jax.nn.sigmoid(a1) + jax.nn.sigmoid(a2)


def _full(arr):
    nd = arr.ndim
    return pl.BlockSpec(arr.shape, lambda i, _n=nd: (0,) * _n)


def _row(last):
    return pl.BlockSpec((_R, last), lambda i: (i, 0))


def _sds(shape):
    return jax.ShapeDtypeStruct(shape, jnp.float32)


def kernel(x, adj, params):
    p = params
    b = {k: p[k].reshape(1, -1) for k in p if k.startswith('ae_') and '_b' in k}
    gamma = p['gamma'].reshape(1, 1)
    cl = p['cluster']

    zae, q1, v0 = pl.pallas_call(
        _pre_kernel,
        grid=(_G,),
        in_specs=[_row(512),
                  _full(p['ae_enc_w0']), _full(b['ae_enc_b0']),
                  _full(p['ae_enc_w1']), _full(b['ae_enc_b1']),
                  _full(p['ae_enc_w2']), _full(b['ae_enc_b2']),
                  _full(p['ae_enc_w3']), _full(b['ae_enc_b3']),
                  _full(p['gae_enc_w0']), _full(p['gae_enc_w1']),
                  _full(p['gae_enc_w2']), _full(cl)],
        out_specs=[_row(32), _row(10), _row(32)],
        out_shape=[_sds((_N, 32)), _sds((_N, 10)), _sds((_N, 32))],
    )(x, p['ae_enc_w0'], b['ae_enc_b0'], p['ae_enc_w1'], b['ae_enc_b1'],
      p['ae_enc_w2'], b['ae_enc_b2'], p['ae_enc_w3'], b['ae_enc_b3'],
      p['gae_enc_w0'], p['gae_enc_w1'], p['gae_enc_w2'], cl)

    def cfull(shape):
        return pl.BlockSpec(shape, lambda t_: (0,) * len(shape))

    adj_spec = pl.BlockSpec(
        (_RC, _N), lambda t_: (jnp.where(t_ < _TC, t_, _TC - 1), 0))

    mega_ins = [v0, zae, p['a'], gamma]
    zs, zt, t3 = pl.pallas_call(
        _mega_kernel,
        grid=(_T,),
        in_specs=[adj_spec] + [cfull(t.shape) for t in mega_ins],
        out_specs=[cfull((_N, 32)), cfull((_N, 32)), cfull((_N, 32))],
        out_shape=[_sds((_N, 32)), _sds((_N, 32)), _sds((_N, 32))],
        scratch_shapes=[pltpu.VMEM((_N, _N), jnp.bfloat16),
                        pltpu.VMEM((_N, 32), jnp.float32),
                        pltpu.VMEM((_N, 32), jnp.float32)],
    )(adj, *mega_ins)

    xhat, zhat, q, q2, tp = pl.pallas_call(
        _tail_kernel,
        grid=(_G,),
        in_specs=[_row(32), _row(32), _row(32),
                  _full(p['ae_dec_w0']), _full(b['ae_dec_b0']),
                  _full(p['ae_dec_w1']), _full(b['ae_dec_b1']),
                  _full(p['ae_dec_w2']), _full(b['ae_dec_b2']),
                  _full(p['ae_dec_w3']), _full(b['ae_dec_b3']),
                  _full(p['gae_dec_w0']), _full(p['gae_dec_w1']),
                  _full(p['gae_dec_w2']), _full(cl)],
        out_specs=[_row(512), _row(512), _row(10), _row(10), _row(32)],
        out_shape=[_sds((_N, 512)), _sds((_N, 512)), _sds((_N, 10)),
                   _sds((_N, 10)), _sds((_N, 32))],
    )(zt, t3, zs,
      p['ae_dec_w0'], b['ae_dec_b0'], p['ae_dec_w1'], b['ae_dec_b1'],
      p['ae_dec_w2'], b['ae_dec_b2'], p['ae_dec_w3'], b['ae_dec_b3'],
      p['gae_dec_w0'], p['gae_dec_w1'], p['gae_dec_w2'], cl)

    adj_hat = pl.pallas_call(
        _adjhat_kernel,
        grid=(_G,),
        in_specs=[_row(32), _full(zs), _row(32), _full(t3)],
        out_specs=_row(_N),
        out_shape=_sds((_N, _N)),
    )(zs, zs, tp, t3)

    return (xhat, zhat, adj_hat, zae, zs, q, q1, q2, zt)


# speedup vs baseline: 1.5110x; 1.0757x over previous
import jax
import jax.numpy as jnp
from jax.experimental import pallas as pl
from jax.experimental.pallas import tpu as pltpu

_N = 4096
_R = 512
_G = _N // _R
_RC = 128
_TC = _N // _RC
_RS = 512
_GS = _N // _RS
_T = _TC + 8 * _GS


def _leaky(z):
    return jnp.where(z >= 0, z, 0.2 * z)


def _dot(a, b):
    return jnp.dot(a, b, preferred_element_type=jnp.float32)


def _soft_assign(z, cluster):
    zn = jnp.sum(z * z, axis=1, keepdims=True)
    cn = jnp.sum(cluster * cluster, axis=1)[None, :]
    d2 = zn + cn - 2.0 * _dot(z, cluster.T)
    q = 1.0 / (1.0 + d2)
    return q / jnp.sum(q, axis=1, keepdims=True)


def _pre_kernel(x_ref, w0, b0, w1, b1, w2, b2, w3, b3,
                gw0, gw1, gw2, cl, zae_out, q1_out, v0_out):
    x = x_ref[...]
    z = _leaky(_dot(x, w0[...]) + b0[...])
    z = _leaky(_dot(z, w1[...]) + b1[...])
    z = _leaky(_dot(z, w2[...]) + b2[...])
    zae = _dot(z, w3[...]) + b3[...]
    zae_out[...] = zae
    q1_out[...] = _soft_assign(zae, cl[...])
    wg = _dot(_dot(gw0[...], gw1[...]), gw2[...])
    v0_out[...] = _dot(x, wg)


def _mega_kernel(adj_ref, v0_ref, zae_ref, a_ref, gamma_ref,
                 zs_out, zt_out, t3_out,
                 adjv, va, vb):
    t = pl.program_id(0)
    s = (t - _TC) // _GS
    r = (t - _TC) % _GS
    rs = pl.ds(r * _RS, _RS)

    @pl.when(t < _TC)
    def _():
        cs = pl.ds(t * _RC, _RC)
        adjv[cs, :] = adj_ref[...].astype(jnp.bfloat16)
        va[cs, :] = v0_ref[cs, :]

    @pl.when(s == 0)
    def _():
        vb[rs, :] = _dot(adjv[rs, :], va[...].astype(jnp.bfloat16))

    @pl.when(s == 1)
    def _():
        va[rs, :] = _dot(adjv[rs, :], vb[...].astype(jnp.bfloat16))

    @pl.when(s == 2)
    def _():
        zs_r = _dot(adjv[rs, :], va[...].astype(jnp.bfloat16))
        zs_out[rs, :] = zs_r
        a_r = a_ref[rs, :]
        vb[rs, :] = a_r * zae_ref[rs, :] + (1.0 - a_r) * zs_r

    @pl.when(s == 3)
    def _():
        va[rs, :] = _dot(adjv[rs, :], vb[...].astype(jnp.bfloat16))

    @pl.when(s == 4)
    def _():
        zl_r = va[rs, :]
        m = jnp.full((_RS, 1), -jnp.inf, dtype=jnp.float32)
        den = jnp.zeros((_RS, 1), dtype=jnp.float32)
        acc = jnp.zeros((_RS, 32), dtype=jnp.float32)
        for c in range(16):
            zl_c = va[pl.ds(c * (_N // 16), _N // 16), :]
            sc = _dot(zl_r, zl_c.T)
            m_new = jnp.maximum(m, jnp.max(sc, axis=1, keepdims=True))
            alpha = jnp.exp(m - m_new)
            pch = jnp.exp(sc - m_new)
            den = den * alpha + jnp.sum(pch, axis=1, keepdims=True)
            acc = acc * alpha + _dot(pch, zl_c)
            m = m_new
        zt_r = gamma_ref[0, 0] * (acc / den) + zl_r
        zt_out[rs, :] = zt_r
        vb[rs, :] = zt_r

    @pl.when(s == 5)
    def _():
        va[rs, :] = _dot(adjv[rs, :], vb[...].astype(jnp.bfloat16))

    @pl.when(s == 6)
    def _():
        vb[rs, :] = _dot(adjv[rs, :], va[...].astype(jnp.bfloat16))

    @pl.when(s == 7)
    def _():
        t3_out[rs, :] = _dot(adjv[rs, :], vb[...].astype(jnp.bfloat16))


def _tail_kernel(zt_ref, t3_ref, zs_ref, zsf_ref, t3f_ref,
                 dw0, db0, dw1, db1, dw2, db2, dw3, db3,
                 gw0, gw1, gw2, cl,
                 xhat_out, zhat_out, q_out, q2_out, ah_out):
    zt = zt_ref[...]
    d = _leaky(_dot(zt, dw0[...]) + db0[...])
    d = _leaky(_dot(d, dw1[...]) + db1[...])
    d = _leaky(_dot(d, dw2[...]) + db2[...])
    xhat_out[...] = _dot(d, dw3[...]) + db3[...]
    ug = _dot(_dot(gw0[...], gw1[...]), gw2[...])
    t3 = t3_ref[...]
    zhat_out[...] = _dot(t3, ug)
    q_out[...] = _soft_assign(zt, cl[...])
    q2_out[...] = _soft_assign(zs_ref[...], cl[...])
    tp = _dot(t3, _dot(ug, ug.T))
    a1 = _dot(zs_ref[...], zsf_ref[...].T)
    a2 = _dot(tp, t3f_ref[...].T)
    ah_out[...] = 1.0 + 0.5 * (jnp.tanh(0.5 * a1) + jnp.tanh(0.5 * a2))


def _full(arr):
    nd = arr.ndim
    return pl.BlockSpec(arr.shape, lambda i, _n=nd: (0,) * _n)


def _row(last):
    return pl.BlockSpec((_R, last), lambda i: (i, 0))


def _sds(shape):
    return jax.ShapeDtypeStruct(shape, jnp.float32)


def kernel(x, adj, params):
    p = params
    b = {k: p[k].reshape(1, -1) for k in p if k.startswith('ae_') and '_b' in k}
    gamma = p['gamma'].reshape(1, 1)
    cl = p['cluster']

    zae, q1, v0 = pl.pallas_call(
        _pre_kernel,
        grid=(_G,),
        in_specs=[_row(512),
                  _full(p['ae_enc_w0']), _full(b['ae_enc_b0']),
                  _full(p['ae_enc_w1']), _full(b['ae_enc_b1']),
                  _full(p['ae_enc_w2']), _full(b['ae_enc_b2']),
                  _full(p['ae_enc_w3']), _full(b['ae_enc_b3']),
                  _full(p['gae_enc_w0']), _full(p['gae_enc_w1']),
                  _full(p['gae_enc_w2']), _full(cl)],
        out_specs=[_row(32), _row(10), _row(32)],
        out_shape=[_sds((_N, 32)), _sds((_N, 10)), _sds((_N, 32))],
    )(x, p['ae_enc_w0'], b['ae_enc_b0'], p['ae_enc_w1'], b['ae_enc_b1'],
      p['ae_enc_w2'], b['ae_enc_b2'], p['ae_enc_w3'], b['ae_enc_b3'],
      p['gae_enc_w0'], p['gae_enc_w1'], p['gae_enc_w2'], cl)

    def cfull(shape):
        return pl.BlockSpec(shape, lambda t_: (0,) * len(shape))

    adj_spec = pl.BlockSpec(
        (_RC, _N), lambda t_: (jnp.where(t_ < _TC, t_, _TC - 1), 0))

    mega_ins = [v0, zae, p['a'], gamma]
    zs, zt, t3 = pl.pallas_call(
        _mega_kernel,
        grid=(_T,),
        in_specs=[adj_spec] + [cfull(t.shape) for t in mega_ins],
        out_specs=[cfull((_N, 32)), cfull((_N, 32)), cfull((_N, 32))],
        out_shape=[_sds((_N, 32)), _sds((_N, 32)), _sds((_N, 32))],
        scratch_shapes=[pltpu.VMEM((_N, _N), jnp.bfloat16),
                        pltpu.VMEM((_N, 32), jnp.float32),
                        pltpu.VMEM((_N, 32), jnp.float32)],
    )(adj, *mega_ins)

    xhat, zhat, q, q2, adj_hat = pl.pallas_call(
        _tail_kernel,
        grid=(_G,),
        in_specs=[_row(32), _row(32), _row(32), _full(zs), _full(t3),
                  _full(p['ae_dec_w0']), _full(b['ae_dec_b0']),
                  _full(p['ae_dec_w1']), _full(b['ae_dec_b1']),
                  _full(p['ae_dec_w2']), _full(b['ae_dec_b2']),
                  _full(p['ae_dec_w3']), _full(b['ae_dec_b3']),
                  _full(p['gae_dec_w0']), _full(p['gae_dec_w1']),
                  _full(p['gae_dec_w2']), _full(cl)],
        out_specs=[_row(512), _row(512), _row(10), _row(10), _row(_N)],
        out_shape=[_sds((_N, 512)), _sds((_N, 512)), _sds((_N, 10)),
                   _sds((_N, 10)), _sds((_N, _N))],
    )(zt, t3, zs, zs, t3,
      p['ae_dec_w0'], b['ae_dec_b0'], p['ae_dec_w1'], b['ae_dec_b1'],
      p['ae_dec_w2'], b['ae_dec_b2'], p['ae_dec_w3'], b['ae_dec_b3'],
      p['gae_dec_w0'], p['gae_dec_w1'], p['gae_dec_w2'], cl)

    return (xhat, zhat, adj_hat, zae, zs, q, q1, q2, zt)


# speedup vs baseline: 1.5995x; 1.0586x over previous
import jax
import jax.numpy as jnp
from jax.experimental import pallas as pl
from jax.experimental.pallas import tpu as pltpu

_N = 4096
_R = 512
_G = _N // _R
_RC = 128
_TC = _N // _RC
_RP = 1024
_RS = 512
_T = _TC + 7 * (_N // _RP) + _N // _RS


def _leaky(z):
    return jnp.where(z >= 0, z, 0.2 * z)


def _dot(a, b):
    return jnp.dot(a, b, preferred_element_type=jnp.float32)


def _soft_assign(z, cluster):
    zn = jnp.sum(z * z, axis=1, keepdims=True)
    cn = jnp.sum(cluster * cluster, axis=1)[None, :]
    d2 = zn + cn - 2.0 * _dot(z, cluster.T)
    q = 1.0 / (1.0 + d2)
    return q / jnp.sum(q, axis=1, keepdims=True)


def _pre_kernel(x_ref, w0, b0, w1, b1, w2, b2, w3, b3,
                gw0, gw1, gw2, cl, zae_out, q1_out, v0_out):
    x = x_ref[...]
    z = _leaky(_dot(x, w0[...]) + b0[...])
    z = _leaky(_dot(z, w1[...]) + b1[...])
    z = _leaky(_dot(z, w2[...]) + b2[...])
    zae = _dot(z, w3[...]) + b3[...]
    zae_out[...] = zae
    q1_out[...] = _soft_assign(zae, cl[...])
    wg = _dot(_dot(gw0[...], gw1[...]), gw2[...])
    v0_out[...] = _dot(x, wg)


def _mega_kernel(adj_ref, v0_ref, zae_ref, a_ref, gamma_ref,
                 zs_out, zt_out, t3_out,
                 adjv, va, vb):
    t = pl.program_id(0)

    @pl.when(t < _TC)
    def _():
        cs = pl.ds(t * _RC, _RC)
        adjv[cs, :] = adj_ref[...].astype(jnp.bfloat16)
        va[cs, :] = v0_ref[cs, :]

    def _pass(lo, dst, src):
        @pl.when(jnp.logical_and(t >= lo, t < lo + _N // _RP))
        def _():
            rr = pl.ds((t - lo) * _RP, _RP)
            dst[rr, :] = _dot(adjv[rr, :], src[...].astype(jnp.bfloat16))

    _pass(_TC, vb, va)
    _pass(_TC + 4, va, vb)

    @pl.when(jnp.logical_and(t >= _TC + 8, t < _TC + 12))
    def _():
        rr = pl.ds((t - (_TC + 8)) * _RP, _RP)
        zs_r = _dot(adjv[rr, :], va[...].astype(jnp.bfloat16))
        zs_out[rr, :] = zs_r
        a_r = a_ref[rr, :]
        vb[rr, :] = a_r * zae_ref[rr, :] + (1.0 - a_r) * zs_r

    _pass(_TC + 12, va, vb)

    @pl.when(jnp.logical_and(t >= _TC + 16, t < _TC + 24))
    def _():
        rs = pl.ds((t - (_TC + 16)) * _RS, _RS)
        zl_r = va[rs, :]
        m = jnp.full((_RS, 1), -jnp.inf, dtype=jnp.float32)
        den = jnp.zeros((_RS, 1), dtype=jnp.float32)
        acc = jnp.zeros((_RS, 32), dtype=jnp.float32)
        for c in range(16):
            zl_c = va[pl.ds(c * (_N // 16), _N // 16), :]
            sc = _dot(zl_r, zl_c.T)
            m_new = jnp.maximum(m, jnp.max(sc, axis=1, keepdims=True))
            alpha = jnp.exp(m - m_new)
            pch = jnp.exp(sc - m_new)
            den = den * alpha + jnp.sum(pch, axis=1, keepdims=True)
            acc = acc * alpha + _dot(pch, zl_c)
            m = m_new
        zt_r = gamma_ref[0, 0] * (acc / den) + zl_r
        zt_out[rs, :] = zt_r
        vb[rs, :] = zt_r

    _pass(_TC + 24, va, vb)
    _pass(_TC + 28, vb, va)

    @pl.when(t >= _TC + 32)
    def _():
        rr = pl.ds((t - (_TC + 32)) * _RP, _RP)
        t3_out[rr, :] = _dot(adjv[rr, :], vb[...].astype(jnp.bfloat16))


def _tail_kernel(zt_ref, t3_ref, zs_ref, zsf_ref, t3f_ref,
                 dw0, db0, dw1, db1, dw2, db2, dw3, db3,
                 gw0, gw1, gw2, cl,
                 xhat_out, zhat_out, q_out, q2_out, ah_out):
    zt = zt_ref[...]
    d = _leaky(_dot(zt, dw0[...]) + db0[...])
    d = _leaky(_dot(d, dw1[...]) + db1[...])
    d = _leaky(_dot(d, dw2[...]) + db2[...])
    xhat_out[...] = _dot(d, dw3[...]) + db3[...]
    ug = _dot(_dot(gw0[...], gw1[...]), gw2[...])
    t3 = t3_ref[...]
    zhat_out[...] = _dot(t3, ug)
    q_out[...] = _soft_assign(zt, cl[...])
    q2_out[...] = _soft_assign(zs_ref[...], cl[...])
    tp = _dot(t3, _dot(ug, ug.T))
    a1 = _dot(zs_ref[...], zsf_ref[...].T)
    a2 = _dot(tp, t3f_ref[...].T)
    ah_out[...] = 1.0 + 0.5 * (jnp.tanh(0.5 * a1) + jnp.tanh(0.5 * a2))


def _full(arr):
    nd = arr.ndim
    return pl.BlockSpec(arr.shape, lambda i, _n=nd: (0,) * _n)


def _row(last):
    return pl.BlockSpec((_R, last), lambda i: (i, 0))


def _sds(shape):
    return jax.ShapeDtypeStruct(shape, jnp.float32)


def kernel(x, adj, params):
    p = params
    b = {k: p[k].reshape(1, -1) for k in p if k.startswith('ae_') and '_b' in k}
    gamma = p['gamma'].reshape(1, 1)
    cl = p['cluster']

    zae, q1, v0 = pl.pallas_call(
        _pre_kernel,
        grid=(_G,),
        in_specs=[_row(512),
                  _full(p['ae_enc_w0']), _full(b['ae_enc_b0']),
                  _full(p['ae_enc_w1']), _full(b['ae_enc_b1']),
                  _full(p['ae_enc_w2']), _full(b['ae_enc_b2']),
                  _full(p['ae_enc_w3']), _full(b['ae_enc_b3']),
                  _full(p['gae_enc_w0']), _full(p['gae_enc_w1']),
                  _full(p['gae_enc_w2']), _full(cl)],
        out_specs=[_row(32), _row(10), _row(32)],
        out_shape=[_sds((_N, 32)), _sds((_N, 10)), _sds((_N, 32))],
    )(x, p['ae_enc_w0'], b['ae_enc_b0'], p['ae_enc_w1'], b['ae_enc_b1'],
      p['ae_enc_w2'], b['ae_enc_b2'], p['ae_enc_w3'], b['ae_enc_b3'],
      p['gae_enc_w0'], p['gae_enc_w1'], p['gae_enc_w2'], cl)

    def cfull(shape):
        return pl.BlockSpec(shape, lambda t_: (0,) * len(shape))

    adj_spec = pl.BlockSpec(
        (_RC, _N), lambda t_: (jnp.where(t_ < _TC, t_, _TC - 1), 0))

    mega_ins = [v0, zae, p['a'], gamma]
    zs, zt, t3 = pl.pallas_call(
        _mega_kernel,
        grid=(_T,),
        in_specs=[adj_spec] + [cfull(t.shape) for t in mega_ins],
        out_specs=[cfull((_N, 32)), cfull((_N, 32)), cfull((_N, 32))],
        out_shape=[_sds((_N, 32)), _sds((_N, 32)), _sds((_N, 32))],
        scratch_shapes=[pltpu.VMEM((_N, _N), jnp.bfloat16),
                        pltpu.VMEM((_N, 32), jnp.float32),
                        pltpu.VMEM((_N, 32), jnp.float32)],
    )(adj, *mega_ins)

    xhat, zhat, q, q2, adj_hat = pl.pallas_call(
        _tail_kernel,
        grid=(_G,),
        in_specs=[_row(32), _row(32), _row(32), _full(zs), _full(t3),
                  _full(p['ae_dec_w0']), _full(b['ae_dec_b0']),
                  _full(p['ae_dec_w1']), _full(b['ae_dec_b1']),
                  _full(p['ae_dec_w2']), _full(b['ae_dec_b2']),
                  _full(p['ae_dec_w3']), _full(b['ae_dec_b3']),
                  _full(p['gae_dec_w0']), _full(p['gae_dec_w1']),
                  _full(p['gae_dec_w2']), _full(cl)],
        out_specs=[_row(512), _row(512), _row(10), _row(10), _row(_N)],
        out_shape=[_sds((_N, 512)), _sds((_N, 512)), _sds((_N, 10)),
                   _sds((_N, 10)), _sds((_N, _N))],
    )(zt, t3, zs, zs, t3,
      p['ae_dec_w0'], b['ae_dec_b0'], p['ae_dec_w1'], b['ae_dec_b1'],
      p['ae_dec_w2'], b['ae_dec_b2'], p['ae_dec_w3'], b['ae_dec_b3'],
      p['gae_dec_w0'], p['gae_dec_w1'], p['gae_dec_w2'], cl)

    return (xhat, zhat, adj_hat, zae, zs, q, q1, q2, zt)
